# baseline (speedup 1.0000x reference)
import functools

import jax
import jax.numpy as jnp
import numpy as np
from jax import lax
from jax.experimental import pallas as pl
from jax.experimental.pallas import tpu as pltpu
from jax.experimental.pallas import tpu_sc as plsc

B = 16384
D = 13
S = 26
V = 100000
E = 32
NI = S + 1
G = 4
EW = E * G


_NW = 32
_RPW = B * S // _NW
_C = 64
_NCH = _RPW // _C
_NB = 4


def _sc_gather(tbl4, idx4):
    mesh = plsc.VectorSubcoreMesh(core_axis_name="c", subcore_axis_name="s")

    @functools.partial(
        pl.kernel,
        mesh=mesh,
        out_type=jax.ShapeDtypeStruct((B * S, EW), jnp.float32),
        scratch_types=[
            pltpu.VMEM((_NB, _C), jnp.int32),
            pltpu.VMEM((_NB, _C, EW), jnp.float32),
            pltpu.SemaphoreType.DMA((_NB,)),
            pltpu.SemaphoreType.DMA((_NB,)),
            pltpu.SemaphoreType.DMA((_NB,)),
        ],
    )
    def k(tbl, idx, out, idxb, rows, si, sg, sw):
        wid = lax.axis_index("s") * 2 + lax.axis_index("c")
        r0 = wid * _RPW

        def fire_idx(c, s):
            pltpu.async_copy(
                idx.at[pl.ds(r0 + c * _C, _C)], idxb.at[s], si.at[s])

        def wait_idx(s):
            pltpu.make_async_copy(
                idx.at[pl.ds(0, _C)], idxb.at[s], si.at[s]).wait()

        def fire_gather(s):
            pltpu.async_copy(tbl.at[idxb.at[s]], rows.at[s], sg.at[s])

        def wait_gather(s):
            pltpu.make_async_copy(
                tbl.at[pl.ds(0, _C)], rows.at[s], sg.at[s]).wait()

        def fire_wb(c, s):
            pltpu.async_copy(
                rows.at[s], out.at[pl.ds(r0 + c * _C, _C)], sw.at[s])

        def wait_wb(s):
            pltpu.make_async_copy(
                rows.at[s], out.at[pl.ds(0, _C)], sw.at[s]).wait()

        for j in range(_NB):
            fire_idx(j, j)
        wait_idx(0)
        fire_gather(0)

        def body(c, carry):
            s = lax.rem(c, _NB)
            sn = lax.rem(c + 1, _NB)

            @pl.when(c + 1 < _NCH)
            def _():
                wait_idx(sn)

                @pl.when(c + 1 >= _NB)
                def _():
                    wait_wb(sn)

                fire_gather(sn)

            wait_gather(s)
            fire_wb(c, s)

            @pl.when(c + _NB < _NCH)
            def _():
                fire_idx(c + _NB, s)

            return carry

        lax.fori_loop(0, _NCH, body, 0, unroll=False)
        for j in range(_NB):
            wait_wb(j)

    return k(tbl4, idx4)


_BB = 512


def _tc_body(dense_ref, sp_ref, sel_ref, wb0, bb0, wb1, bb1, wb2, bb2,
             w0d, w0z, bt0, wt1, bt1, wt2, bt2, out_ref):
    f32 = jnp.float32
    x = dense_ref[...].astype(jnp.bfloat16)
    h = jnp.maximum(jnp.dot(x, wb0[...], preferred_element_type=f32)
                    + bb0[...], 0.0)
    h = jnp.maximum(jnp.dot(h.astype(jnp.bfloat16), wb1[...],
                            preferred_element_type=f32) + bb1[...], 0.0)
    de = (jnp.dot(h.astype(jnp.bfloat16), wb2[...],
                  preferred_element_type=f32) + bb2[...])
    sp = sp_ref[...]
    sel = sel_ref[...]
    emb = jnp.zeros((_BB * S, E), f32)
    for g in range(G):
        emb = emb + jnp.where(sel == g, sp[:, g * E:(g + 1) * E], 0.0)
    sp3 = emb.reshape(_BB, S, E)
    t3 = jnp.concatenate([de.reshape(_BB, 1, E).astype(jnp.bfloat16),
                          sp3.astype(jnp.bfloat16)], axis=1)
    z3 = lax.dot_general(t3, t3, (((2,), (2,)), ((0,), (0,))),
                         preferred_element_type=f32)
    zf = z3.reshape(_BB, NI * NI)
    t0 = (jnp.dot(de.astype(jnp.bfloat16), w0d[...],
                  preferred_element_type=f32)
          + jnp.dot(zf.astype(jnp.bfloat16), w0z[...],
                    preferred_element_type=f32) + bt0[...])
    t0 = jnp.maximum(t0, 0.0)
    t1 = jnp.maximum(jnp.dot(t0.astype(jnp.bfloat16), wt1[...],
                             preferred_element_type=f32) + bt1[...], 0.0)
    o = jnp.dot(t1.astype(jnp.bfloat16), wt2[...],
                preferred_element_type=f32) + bt2[...]
    out_ref[...] = jax.nn.sigmoid(o)


def _tc_forward(dense_x, sparse_emb, sel, wb0, bb0, wb1, bb1, wb2, bb2,
                w0d, w0z, bt0, wt1, bt1, wt2, bt2):
    n_blk = B // _BB
    full = lambda s: pl.BlockSpec(s, lambda i: (0,) * len(s))
    grid_spec = pl.GridSpec(
        grid=(n_blk,),
        in_specs=[
            pl.BlockSpec((_BB, D), lambda i: (i, 0)),
            pl.BlockSpec((_BB * S, EW), lambda i: (i, 0)),
            pl.BlockSpec((_BB * S, 1), lambda i: (i, 0)),
            full(wb0.shape), full(bb0.shape), full(wb1.shape),
            full(bb1.shape), full(wb2.shape), full(bb2.shape),
            full(w0d.shape), full(w0z.shape), full(bt0.shape),
            full(wt1.shape), full(bt1.shape), full(wt2.shape),
            full(bt2.shape),
        ],
        out_specs=pl.BlockSpec((_BB, 1), lambda i: (i, 0)),
    )
    return pl.pallas_call(
        _tc_body,
        grid_spec=grid_spec,
        out_shape=jax.ShapeDtypeStruct((B, 1), jnp.float32),
    )(dense_x, sparse_emb, sel, wb0, bb0, wb1, bb1, wb2, bb2,
      w0d, w0z, bt0, wt1, bt1, wt2, bt2)


def kernel(dense_x, sparse_x, emb_tables, Wb0, bb0, Wb1, bb1, Wb2, bb2,
           Wt0, bt0, Wt1, bt1, Wt2, bt2):
    bf16 = jnp.bfloat16
    offs = (jnp.arange(S, dtype=jnp.int32) * V)[None, :]
    idx_flat = (sparse_x.astype(jnp.int32) + offs).reshape(B * S)
    sel = (idx_flat & (G - 1)).reshape(B * S, 1)
    tbl4 = emb_tables.reshape(S * V // G, EW)
    sparse_emb = _sc_gather(tbl4, idx_flat >> 2)

    li, lj = np.tril_indices(NI, -1)
    rows = jnp.asarray(li * NI + lj, dtype=jnp.int32)
    w0z = jnp.zeros((NI * NI, Wt0.shape[1]), jnp.float32).at[rows].set(Wt0[E:])
    w0d = Wt0[:E]

    return _tc_forward(
        dense_x, sparse_emb, sel,
        Wb0.astype(bf16), bb0, Wb1.astype(bf16), bb1, Wb2.astype(bf16), bb2,
        w0d.astype(bf16), w0z.astype(bf16), bt0,
        Wt1.astype(bf16), bt1, Wt2.astype(bf16), bt2)


# speedup vs baseline: 1.2807x; 1.2807x over previous
import functools

import jax
import jax.numpy as jnp
import numpy as np
from jax import lax
from jax.experimental import pallas as pl
from jax.experimental.pallas import tpu as pltpu
from jax.experimental.pallas import tpu_sc as plsc

B = 16384
D = 13
S = 26
V = 100000
E = 32
NI = S + 1


_NW = 32
_RPW = B * S // _NW
_C = 128
_NCH = _RPW // _C
_NB = 4


def _sc_gather(tbl_flat, idx_flat):
    mesh = plsc.VectorSubcoreMesh(core_axis_name="c", subcore_axis_name="s")

    @functools.partial(
        pl.kernel,
        mesh=mesh,
        out_type=jax.ShapeDtypeStruct((B * S, E), jnp.float32),
        compiler_params=pltpu.CompilerParams(use_tc_tiling_on_sc=False),
    scratch_types=[
            pltpu.VMEM((_NB, _C), jnp.int32),
            pltpu.VMEM((_NB, _C, E), jnp.float32),
            pltpu.SemaphoreType.DMA((_NB,)),
            pltpu.SemaphoreType.DMA((_NB,)),
            pltpu.SemaphoreType.DMA((_NB,)),
        ],
    )
    def k(tbl, idx, out, idxb, rows, si, sg, sw):
        wid = lax.axis_index("s") * 2 + lax.axis_index("c")
        r0 = wid * _RPW

        def fire_idx(c, s):
            pltpu.async_copy(
                idx.at[pl.ds(r0 + c * _C, _C)], idxb.at[s], si.at[s])

        def wait_idx(s):
            pltpu.make_async_copy(
                idx.at[pl.ds(0, _C)], idxb.at[s], si.at[s]).wait()

        def fire_gather(s):
            pltpu.async_copy(tbl.at[idxb.at[s]], rows.at[s], sg.at[s])

        def wait_gather(s):
            pltpu.make_async_copy(
                tbl.at[pl.ds(0, _C)], rows.at[s], sg.at[s]).wait()

        def fire_wb(c, s):
            pltpu.async_copy(
                rows.at[s], out.at[pl.ds(r0 + c * _C, _C)], sw.at[s])

        def wait_wb(s):
            pltpu.make_async_copy(
                rows.at[s], out.at[pl.ds(0, _C)], sw.at[s]).wait()

        for j in range(_NB):
            fire_idx(j, j)
        wait_idx(0)
        fire_gather(0)

        def body(c, carry):
            s = lax.rem(c, _NB)
            sn = lax.rem(c + 1, _NB)

            @pl.when(c + 1 < _NCH)
            def _():
                wait_idx(sn)

                @pl.when(c + 1 >= _NB)
                def _():
                    wait_wb(sn)

                fire_gather(sn)

            wait_gather(s)
            fire_wb(c, s)

            @pl.when(c + _NB < _NCH)
            def _():
                fire_idx(c + _NB, s)

            return carry

        lax.fori_loop(0, _NCH, body, 0, unroll=False)
        for j in range(_NB):
            wait_wb(j)

    return k(tbl_flat, idx_flat)


_BB = 512


def _tc_body(dense_ref, sp_ref, wb0, bb0, wb1, bb1, wb2, bb2,
             w0d, w0z, bt0, wt1, bt1, wt2, bt2, out_ref):
    f32 = jnp.float32
    x = dense_ref[...].astype(jnp.bfloat16)
    h = jnp.maximum(jnp.dot(x, wb0[...], preferred_element_type=f32)
                    + bb0[...], 0.0)
    h = jnp.maximum(jnp.dot(h.astype(jnp.bfloat16), wb1[...],
                            preferred_element_type=f32) + bb1[...], 0.0)
    de = (jnp.dot(h.astype(jnp.bfloat16), wb2[...],
                  preferred_element_type=f32) + bb2[...])
    sp3 = sp_ref[...].reshape(_BB, S, E)
    t3 = jnp.concatenate([de.reshape(_BB, 1, E).astype(jnp.bfloat16),
                          sp3.astype(jnp.bfloat16)], axis=1)
    z3 = lax.dot_general(t3, t3, (((2,), (2,)), ((0,), (0,))),
                         preferred_element_type=f32)
    zf = z3.reshape(_BB, NI * NI)
    t0 = (jnp.dot(de.astype(jnp.bfloat16), w0d[...],
                  preferred_element_type=f32)
          + jnp.dot(zf.astype(jnp.bfloat16), w0z[...],
                    preferred_element_type=f32) + bt0[...])
    t0 = jnp.maximum(t0, 0.0)
    t1 = jnp.maximum(jnp.dot(t0.astype(jnp.bfloat16), wt1[...],
                             preferred_element_type=f32) + bt1[...], 0.0)
    o = jnp.dot(t1.astype(jnp.bfloat16), wt2[...],
                preferred_element_type=f32) + bt2[...]
    out_ref[...] = jax.nn.sigmoid(o)


def _tc_forward(dense_x, sparse_emb, wb0, bb0, wb1, bb1, wb2, bb2,
                w0d, w0z, bt0, wt1, bt1, wt2, bt2):
    n_blk = B // _BB
    full = lambda s: pl.BlockSpec(s, lambda i: (0,) * len(s))
    grid_spec = pl.GridSpec(
        grid=(n_blk,),
        in_specs=[
            pl.BlockSpec((_BB, D), lambda i: (i, 0)),
            pl.BlockSpec((_BB * S, E), lambda i: (i, 0)),
            full(wb0.shape), full(bb0.shape), full(wb1.shape),
            full(bb1.shape), full(wb2.shape), full(bb2.shape),
            full(w0d.shape), full(w0z.shape), full(bt0.shape),
            full(wt1.shape), full(bt1.shape), full(wt2.shape),
            full(bt2.shape),
        ],
        out_specs=pl.BlockSpec((_BB, 1), lambda i: (i, 0)),
    )
    return pl.pallas_call(
        _tc_body,
        grid_spec=grid_spec,
        out_shape=jax.ShapeDtypeStruct((B, 1), jnp.float32),
    )(dense_x, sparse_emb, wb0, bb0, wb1, bb1, wb2, bb2,
      w0d, w0z, bt0, wt1, bt1, wt2, bt2)


def kernel(dense_x, sparse_x, emb_tables, Wb0, bb0, Wb1, bb1, Wb2, bb2,
           Wt0, bt0, Wt1, bt1, Wt2, bt2):
    bf16 = jnp.bfloat16
    offs = (jnp.arange(S, dtype=jnp.int32) * V)[None, :]
    idx_flat = (sparse_x.astype(jnp.int32) + offs).reshape(B * S)
    sparse_emb = _sc_gather(emb_tables.reshape(S * V, E), idx_flat)

    li, lj = np.tril_indices(NI, -1)
    P = np.zeros((NI * NI, len(li)), np.float32)
    P[li * NI + lj, np.arange(len(li))] = 1.0
    w0z = jnp.dot(jnp.asarray(P), Wt0[E:])
    w0d = Wt0[:E]

    return _tc_forward(
        dense_x, sparse_emb,
        Wb0.astype(bf16), bb0, Wb1.astype(bf16), bb1, Wb2.astype(bf16), bb2,
        w0d.astype(bf16), w0z.astype(bf16), bt0,
        Wt1.astype(bf16), bt1, Wt2.astype(bf16), bt2)
